# baseline (speedup 1.0000x reference)
import functools

import jax
import jax.numpy as jnp
from jax.experimental import pallas as pl
from jax.experimental.pallas import tpu as pltpu


def _linear_kernel(x_ref, w_ref, b_ref, o_ref):
    acc = jax.lax.dot_general(
        x_ref[...], w_ref[...],
        dimension_numbers=(((1,), (1,)), ((), ())),
        preferred_element_type=jnp.float32,
    )
    o_ref[...] = acc + b_ref[...]


def _spmm_kernel(adj_ref, s_ref, o_ref):
    o_ref[...] = jnp.dot(adj_ref[...], s_ref[...],
                         preferred_element_type=jnp.float32)


def kernel(x, W, b, adj):
    n, d_in = x.shape
    d_out = W.shape[0]
    b2 = b.reshape(1, d_out)

    mb1 = 2000 if n % 2000 == 0 else n
    support = pl.pallas_call(
        _linear_kernel,
        grid=(n // mb1,),
        in_specs=[
            pl.BlockSpec((mb1, d_in), lambda i: (i, 0)),
            pl.BlockSpec((d_out, d_in), lambda i: (0, 0)),
            pl.BlockSpec((1, d_out), lambda i: (0, 0)),
        ],
        out_specs=pl.BlockSpec((mb1, d_out), lambda i: (i, 0)),
        out_shape=jax.ShapeDtypeStruct((n, d_out), jnp.float32),
    )(x, W, b2)

    mb = 400 if n % 400 == 0 else n
    nm = n // mb
    out = pl.pallas_call(
        _spmm_kernel,
        grid=(nm,),
        in_specs=[
            pl.BlockSpec((mb, n), lambda i: (i, 0)),
            pl.BlockSpec((n, d_out), lambda i: (0, 0)),
        ],
        out_specs=pl.BlockSpec((mb, d_out), lambda i: (i, 0)),
        out_shape=jax.ShapeDtypeStruct((n, d_out), jnp.float32),
        compiler_params=pltpu.CompilerParams(
            dimension_semantics=("parallel",),
        ),
    )(adj, support)
    return out


# speedup vs baseline: 1.0569x; 1.0569x over previous
import jax
import jax.numpy as jnp
from jax.experimental import pallas as pl
from jax.experimental.pallas import tpu as pltpu


def _fused_kernel(adj_ref, x_ref, w_ref, b_ref, o_ref, s_ref):
    @pl.when(pl.program_id(0) == 0)
    def _compute_support():
        s_ref[...] = jax.lax.dot_general(
            x_ref[...], w_ref[...],
            dimension_numbers=(((1,), (1,)), ((), ())),
            preferred_element_type=jnp.float32,
        ) + b_ref[...]

    o_ref[...] = jnp.dot(adj_ref[...], s_ref[...],
                         preferred_element_type=jnp.float32)


def kernel(x, W, b, adj):
    n, d_in = x.shape
    d_out = W.shape[0]
    b2 = b.reshape(1, d_out)

    mb = 400 if n % 400 == 0 else n
    nm = n // mb
    out = pl.pallas_call(
        _fused_kernel,
        grid=(nm,),
        in_specs=[
            pl.BlockSpec((mb, n), lambda i: (i, 0)),
            pl.BlockSpec((n, d_in), lambda i: (0, 0)),
            pl.BlockSpec((d_out, d_in), lambda i: (0, 0)),
            pl.BlockSpec((1, d_out), lambda i: (0, 0)),
        ],
        out_specs=pl.BlockSpec((mb, d_out), lambda i: (i, 0)),
        out_shape=jax.ShapeDtypeStruct((n, d_out), jnp.float32),
        scratch_shapes=[pltpu.VMEM((n, d_out), jnp.float32)],
        compiler_params=pltpu.CompilerParams(
            dimension_semantics=("arbitrary",),
        ),
    )(adj, x, W, b2)
    return out


# speedup vs baseline: 1.0569x; 1.0000x over previous
import jax
import jax.numpy as jnp
from jax.experimental import pallas as pl
from jax.experimental.pallas import tpu as pltpu


def _fused_kernel(adj_ref, x_ref, w_ref, b_ref, o_ref, s_ref):
    @pl.when(pl.program_id(0) == 0)
    def _compute_support():
        s_ref[...] = jax.lax.dot_general(
            x_ref[...], w_ref[...],
            dimension_numbers=(((1,), (1,)), ((), ())),
            preferred_element_type=jnp.float32,
        ) + b_ref[...]

    o_ref[...] = jnp.dot(adj_ref[...], s_ref[...],
                         preferred_element_type=jnp.float32)


def kernel(x, W, b, adj):
    n, d_in = x.shape
    d_out = W.shape[0]
    b2 = b.reshape(1, d_out)

    mb = 400 if n % 400 == 0 else n
    nm = n // mb
    out = pl.pallas_call(
        _fused_kernel,
        grid=(nm,),
        in_specs=[
            pl.BlockSpec((mb, n), lambda i: (i, 0)),
            pl.BlockSpec((n, d_in), lambda i: (0, 0)),
            pl.BlockSpec((d_out, d_in), lambda i: (0, 0)),
            pl.BlockSpec((1, d_out), lambda i: (0, 0)),
        ],
        out_specs=pl.BlockSpec((mb, d_out), lambda i: (i, 0)),
        out_shape=jax.ShapeDtypeStruct((n, d_out), jnp.float32),
        scratch_shapes=[pltpu.VMEM((n, d_out), jnp.float32)],
        compiler_params=pltpu.CompilerParams(
            dimension_semantics=("arbitrary",),
            vmem_limit_bytes=100_000_000,
        ),
    )(adj, x, W, b2)
    return out
